# baseline (speedup 1.0000x reference)
import functools

import jax
import jax.numpy as jnp
from jax import lax
from jax.experimental import pallas as pl
from jax.experimental.pallas import tpu as pltpu
from jax.experimental.pallas import tpu_sc as plsc

B, S, D, F, E = 4, 2048, 1024, 4096, 8
N = B * S
T = 256
NPAD = N + E * T
NB = NPAD // T
NBE = 48
NC, NS, L = 2, 16, 16
NW = NC * NS
TOK = N // NW
FCH = 1024

_mesh = plsc.VectorSubcoreMesh(core_axis_name="c", subcore_axis_name="s",
                               num_cores=NC, num_subcores=NS)
_sc_params = pltpu.CompilerParams(needs_layout_passes=False)


def _wid():
    return lax.axis_index("s") * NC + lax.axis_index("c")


def _router_body(x_ref, wr_ref, prob_ref, exp_ref, hist_ref):
    logits = lax.dot_general(x_ref[...], wr_ref[...],
                             (((1,), (1,)), ((), ())),
                             preferred_element_type=jnp.float32)
    route = jax.nn.sigmoid(logits)
    m = jnp.max(route, axis=1, keepdims=True)
    ids = lax.broadcasted_iota(jnp.int32, route.shape, 1)
    idx = jnp.min(jnp.where(route == m, ids, E), axis=1, keepdims=True)
    prob_ref[...] = m
    exp_ref[...] = idx
    onehot = (lax.broadcasted_iota(jnp.int32, (512, L), 1) == idx
              ).astype(jnp.int32)
    hist_ref[0, 0:1, :] = jnp.sum(onehot[:TOK], axis=0, keepdims=True)
    hist_ref[0, 1:2, :] = jnp.sum(onehot[TOK:], axis=0, keepdims=True)


def _router(x2d, w_router):
    TBR = 512
    return pl.pallas_call(
        _router_body,
        grid=(N // TBR,),
        in_specs=[
            pl.BlockSpec((TBR, D), lambda i: (i, 0)),
            pl.BlockSpec((E, D), lambda i: (0, 0)),
        ],
        out_specs=[
            pl.BlockSpec((TBR, 1), lambda i: (i, 0)),
            pl.BlockSpec((TBR, 1), lambda i: (i, 0)),
            pl.BlockSpec((1, 2, L), lambda i: (i, 0, 0)),
        ],
        out_shape=[
            jax.ShapeDtypeStruct((N, 1), jnp.float32),
            jax.ShapeDtypeStruct((N, 1), jnp.int32),
            jax.ShapeDtypeStruct((NW // 2, 2, L), jnp.int32),
        ],
    )(x2d, w_router)


def _sc_dispatch_body(exp_hbm, prob_hbm, x_hbm, hist_hbm,
                 xs_hbm, ps_hbm, pos_hbm, be_hbm, act_hbm,
                 evm, hvm, gvm, posflat,
                 i0, i1, i2, i3, i4, i5, i6, i7,
                 xbufa, xbufb, pvm, bevm, actvm,
                 semin_a, semin_b, semout_a, semout_b, semaux):
    wid = _wid()
    tokbase = wid * TOK
    pltpu.sync_copy(exp_hbm.at[pl.ds(tokbase, TOK)], evm)
    pltpu.sync_copy(hist_hbm, hvm)
    pltpu.sync_copy(prob_hbm.at[pl.ds(tokbase, TOK)], pvm)

    lane = jnp.arange(L, dtype=jnp.int32)
    widv = jnp.full((L,), 1, jnp.int32) * wid
    tot = jnp.zeros((L,), jnp.int32)
    bef = jnp.zeros((L,), jnp.int32)
    for t in range(NW):
        row = hvm[t]
        tot = tot + row
        bef = bef + jnp.where(jnp.full((L,), t, jnp.int32) < widv, row, 0)

    padded = ((tot + (T - 1)) >> 8) << 8
    incl = plsc.cumsum(padded)
    mybase = (incl - padded) + bef

    @pl.when(wid == 0)
    def _():
        end_s = jnp.sum(jnp.where(lane == jnp.full((L,), E - 1, jnp.int32),
                                  incl, 0))
        end = jnp.broadcast_to(end_s, (L,))
        for k in range(NBE // L):
            bstart = (jnp.arange(L, dtype=jnp.int32) + L * k) * T
            bcount = jnp.zeros((L,), jnp.int32)
            for e in range(E):
                bnd_s = jnp.sum(jnp.where(lane == jnp.full((L,), e, jnp.int32),
                                          incl, 0))
                bnd = jnp.broadcast_to(bnd_s, (L,))
                bcount = bcount + jnp.where(bstart >= bnd, 1, 0)
            bevm[pl.ds(k * L, L)] = jnp.minimum(bcount, E - 1)
            actvm[pl.ds(k * L, L)] = jnp.where(bstart < end, 1, 0)
        pltpu.sync_copy(bevm, be_hbm)
        pltpu.sync_copy(actvm, act_hbm)

    base = mybase
    idxbufs = [i0, i1, i2, i3, i4, i5, i6, i7]
    for i in range(TOK // L):
        v = evm[pl.ds(i * L, L)]
        gvm[...] = base
        bsel = plsc.load_gather(gvm, [v])
        rank = jnp.zeros((L,), jnp.int32)
        for e in range(E):
            ev = jnp.full((L,), e, jnp.int32)
            m = v == ev
            pref = plsc.cumsum(jnp.where(m, 1, 0))
            rank = jnp.where(m, pref - 1, rank)
            c = jnp.broadcast_to(pref[L - 1], (L,))
            base = base + jnp.where(lane == ev, c, 0)
        posv = bsel + rank
        posflat[pl.ds(i * L, L)] = posv
        idxbufs[i // 2][pl.ds((i % 2) * L, L)] = posv

    aux = [pltpu.async_copy(posflat, pos_hbm.at[pl.ds(tokbase, TOK)], semaux)]
    for j in range(8):
        aux.append(pltpu.async_copy(pvm.at[pl.ds(32 * j, 32)],
                                    ps_hbm.at[idxbufs[j]], semaux))

    bufs = [xbufa, xbufb]
    sin = [semin_a, semin_b]
    sout = [semout_a, semout_b]
    pend_out = [None, None]
    d_in = pltpu.async_copy(x_hbm.at[pl.ds(tokbase, 32)], bufs[0], sin[0])
    for j in range(8):
        b = j % 2
        d_in.wait()
        d_out = pltpu.async_copy(bufs[b], xs_hbm.at[idxbufs[j]], sout[b])
        if j + 1 < 8:
            if pend_out[1 - b] is not None:
                pend_out[1 - b].wait()
            d_in = pltpu.async_copy(
                x_hbm.at[pl.ds(tokbase + 32 * (j + 1), 32)],
                bufs[1 - b], sin[1 - b])
        pend_out[b] = d_out
    pend_out[0].wait()
    pend_out[1].wait()
    for d in aux:
        d.wait()


_DISP_OUT = [
    jax.ShapeDtypeStruct((NPAD, D), jnp.float32),
    jax.ShapeDtypeStruct((NPAD,), jnp.float32),
    jax.ShapeDtypeStruct((N,), jnp.int32),
    jax.ShapeDtypeStruct((NBE,), jnp.int32),
    jax.ShapeDtypeStruct((NBE,), jnp.int32),
]
_DISP_SCRATCH = (
    [
        pltpu.VMEM((TOK,), jnp.int32),
        pltpu.VMEM((NW, L), jnp.int32),
        pltpu.VMEM((L,), jnp.int32),
        pltpu.VMEM((TOK,), jnp.int32),
    ]
    + [pltpu.VMEM((32,), jnp.int32)] * 8
    + [
        pltpu.VMEM((32, D), jnp.float32),
        pltpu.VMEM((32, D), jnp.float32),
        pltpu.VMEM((TOK,), jnp.float32),
        pltpu.VMEM((NBE,), jnp.int32),
        pltpu.VMEM((NBE,), jnp.int32),
        pltpu.SemaphoreType.DMA,
        pltpu.SemaphoreType.DMA,
        pltpu.SemaphoreType.DMA,
        pltpu.SemaphoreType.DMA,
        pltpu.SemaphoreType.DMA,
    ]
)
_sc_dispatch = pl.kernel(_sc_dispatch_body, out_type=_DISP_OUT, mesh=_mesh,
                         compiler_params=_sc_params,
                         scratch_types=_DISP_SCRATCH)


FB = 2048
NF = F // FB


def _glu_partial(x, wg_ref, wu_ref, wd_ref):
    x_bf = x.astype(jnp.bfloat16)
    acc = jnp.zeros((T, D), jnp.float32)
    for c in range(FB // FCH):
        wg = wg_ref[0, pl.ds(c * FCH, FCH), :].astype(jnp.bfloat16)
        wu = wu_ref[0, pl.ds(c * FCH, FCH), :].astype(jnp.bfloat16)
        g = lax.dot_general(x_bf, wg, (((1,), (1,)), ((), ())),
                            preferred_element_type=jnp.float32)
        u = lax.dot_general(x_bf, wu, (((1,), (1,)), ((), ())),
                            preferred_element_type=jnp.float32)
        h = (g * jax.nn.sigmoid(g)) * u
        wd = wd_ref[0, :, pl.ds(c * FCH, FCH)].astype(jnp.bfloat16)
        acc = acc + lax.dot_general(h.astype(jnp.bfloat16), wd,
                                    (((1,), (1,)), ((), ())),
                                    preferred_element_type=jnp.float32)
    return acc


def _moe_body0(be_ref, act_ref, x_ref, wg_ref, wu_ref, wd_ref, p_ref, o_ref):
    @pl.when(act_ref[pl.program_id(0)] == 1)
    def _():
        o_ref[...] = _glu_partial(x_ref[...], wg_ref, wu_ref, wd_ref) \
            * p_ref[...]


def _moe_body1(be_ref, act_ref, x_ref, wg_ref, wu_ref, wd_ref, p_ref,
               part_ref, o_ref):
    @pl.when(act_ref[pl.program_id(0)] == 1)
    def _():
        o_ref[...] = part_ref[...] + \
            _glu_partial(x_ref[...], wg_ref, wu_ref, wd_ref) * p_ref[...]


_moe_params = pltpu.CompilerParams(
    dimension_semantics=("arbitrary",),
    vmem_limit_bytes=64 * 1024 * 1024,
)


def _moe_mlp(block_expert, act, xs, wg, wu, wd, ps2d):
    def wspecs(fi):
        return [
            pl.BlockSpec((1, FB, D), lambda b, s, a: (s[b], fi, 0)),
            pl.BlockSpec((1, FB, D), lambda b, s, a: (s[b], fi, 0)),
            pl.BlockSpec((1, D, FB), lambda b, s, a: (s[b], 0, fi)),
        ]

    xspec = pl.BlockSpec((T, D), lambda b, s, a: (b, 0))
    pspec = pl.BlockSpec((T, 1), lambda b, s, a: (b, 0))
    part = pl.pallas_call(
        _moe_body0,
        grid_spec=pltpu.PrefetchScalarGridSpec(
            num_scalar_prefetch=2,
            grid=(NB,),
            in_specs=[xspec] + wspecs(0) + [pspec],
            out_specs=xspec,
        ),
        out_shape=jax.ShapeDtypeStruct((NPAD, D), jnp.float32),
        compiler_params=_moe_params,
    )(block_expert, act, xs, wg, wu, wd, ps2d)
    return pl.pallas_call(
        _moe_body1,
        grid_spec=pltpu.PrefetchScalarGridSpec(
            num_scalar_prefetch=2,
            grid=(NB,),
            in_specs=[xspec] + wspecs(1) + [pspec, xspec],
            out_specs=xspec,
        ),
        out_shape=jax.ShapeDtypeStruct((NPAD, D), jnp.float32),
        compiler_params=_moe_params,
    )(block_expert, act, xs, wg, wu, wd, ps2d, part)


def _sc_unpermute_body(os_hbm, pos_hbm, y_hbm,
                       i0, i1, i2, i3, i4, i5, i6, i7,
                       posvm, bufa, bufb,
                       semin_a, semin_b, semout_a, semout_b):
    tokbase = _wid() * TOK
    pltpu.sync_copy(pos_hbm.at[pl.ds(tokbase, TOK)], posvm)
    idxbufs = [i0, i1, i2, i3, i4, i5, i6, i7]
    for j in range(8):
        for q in range(2):
            idxbufs[j][pl.ds(q * L, L)] = posvm[pl.ds(32 * j + q * L, L)]
    bufs = [bufa, bufb]
    sin = [semin_a, semin_b]
    sout = [semout_a, semout_b]
    pend_out = [None, None]
    d_in = pltpu.async_copy(os_hbm.at[idxbufs[0]], bufs[0], sin[0])
    for j in range(8):
        b = j % 2
        d_in.wait()
        d_out = pltpu.async_copy(
            bufs[b], y_hbm.at[pl.ds(tokbase + 32 * j, 32)], sout[b])
        if j + 1 < 8:
            if pend_out[1 - b] is not None:
                pend_out[1 - b].wait()
            d_in = pltpu.async_copy(os_hbm.at[idxbufs[j + 1]],
                                    bufs[1 - b], sin[1 - b])
        pend_out[b] = d_out
    pend_out[0].wait()
    pend_out[1].wait()


_UNP_OUT = jax.ShapeDtypeStruct((N, D), jnp.float32)
_UNP_SCRATCH = (
    [pltpu.VMEM((32,), jnp.int32)] * 8
    + [
        pltpu.VMEM((TOK,), jnp.int32),
        pltpu.VMEM((32, D), jnp.float32),
        pltpu.VMEM((32, D), jnp.float32),
        pltpu.SemaphoreType.DMA,
        pltpu.SemaphoreType.DMA,
        pltpu.SemaphoreType.DMA,
        pltpu.SemaphoreType.DMA,
    ]
)
_sc_unpermute = pl.kernel(_sc_unpermute_body, out_type=_UNP_OUT, mesh=_mesh,
                          compiler_params=_sc_params,
                          scratch_types=_UNP_SCRATCH)


def kernel(hidden_states, W_router, W_gate, W_up, W_down):
    x2d = hidden_states.reshape(N, D)
    prob2d, exp2d, hist3 = _router(x2d, W_router)
    expert = exp2d.reshape(N)
    prob = prob2d.reshape(N)

    xs, ps, pos, block_expert, act = _sc_dispatch(
        expert, prob, x2d, hist3.reshape(NW, L))

    out_sorted = _moe_mlp(block_expert, act, xs, W_gate, W_up, W_down,
                          ps.reshape(NPAD, 1))

    y = _sc_unpermute(out_sorted, pos)
    return y.reshape(B, S, D)


# speedup vs baseline: 1.0041x; 1.0041x over previous
import jax
import jax.numpy as jnp
from jax import lax
from jax.experimental import pallas as pl
from jax.experimental.pallas import tpu as pltpu
from jax.experimental.pallas import tpu_sc as plsc

B, S, D, F, E = 4, 2048, 1024, 4096, 8
N = B * S
T = 256
NPAD = N + E * T
NB = NPAD // T
NBE = 48
NC, NS, L = 2, 16, 16
NW = NC * NS
TOK = N // NW
FCH = 1024

_mesh = plsc.VectorSubcoreMesh(core_axis_name="c", subcore_axis_name="s",
                               num_cores=NC, num_subcores=NS)
_sc_params = pltpu.CompilerParams(needs_layout_passes=False)


def _wid():
    return lax.axis_index("s") * NC + lax.axis_index("c")


def _router_body(x_ref, wr_ref, prob_ref, exp_ref, hist_ref):
    logits = lax.dot_general(x_ref[...], wr_ref[...],
                             (((1,), (1,)), ((), ())),
                             preferred_element_type=jnp.float32)
    route = jax.nn.sigmoid(logits)
    m = jnp.max(route, axis=1, keepdims=True)
    ids = lax.broadcasted_iota(jnp.int32, route.shape, 1)
    idx = jnp.min(jnp.where(route == m, ids, E), axis=1, keepdims=True)
    prob_ref[...] = m
    exp_ref[...] = idx
    onehot = (lax.broadcasted_iota(jnp.int32, (512, L), 1) == idx
              ).astype(jnp.int32)
    hist_ref[0, 0:1, :] = jnp.sum(onehot[:TOK], axis=0, keepdims=True)
    hist_ref[0, 1:2, :] = jnp.sum(onehot[TOK:], axis=0, keepdims=True)


def _router(x2d, w_router):
    TBR = 512
    return pl.pallas_call(
        _router_body,
        grid=(N // TBR,),
        in_specs=[
            pl.BlockSpec((TBR, D), lambda i: (i, 0)),
            pl.BlockSpec((E, D), lambda i: (0, 0)),
        ],
        out_specs=[
            pl.BlockSpec((TBR, 1), lambda i: (i, 0)),
            pl.BlockSpec((TBR, 1), lambda i: (i, 0)),
            pl.BlockSpec((1, 2, L), lambda i: (i, 0, 0)),
        ],
        out_shape=[
            jax.ShapeDtypeStruct((N, 1), jnp.float32),
            jax.ShapeDtypeStruct((N, 1), jnp.int32),
            jax.ShapeDtypeStruct((NW // 2, 2, L), jnp.int32),
        ],
    )(x2d, w_router)


def _sc_dispatch_body(exp_hbm, prob_hbm, x_hbm, hist_hbm,
                 xs_hbm, ps_hbm, pos_hbm, be_hbm, act_hbm,
                 evm, hvm, gvm, posflat,
                 i0, i1, i2, i3, i4, i5, i6, i7,
                 xbufa, xbufb, pvm, bevm, actvm,
                 semin_a, semin_b, semout_a, semout_b, semaux):
    wid = _wid()
    tokbase = wid * TOK
    pltpu.sync_copy(exp_hbm.at[pl.ds(tokbase, TOK)], evm)
    pltpu.sync_copy(hist_hbm, hvm)
    pltpu.sync_copy(prob_hbm.at[pl.ds(tokbase, TOK)], pvm)

    lane = jnp.arange(L, dtype=jnp.int32)
    widv = jnp.full((L,), 1, jnp.int32) * wid
    tot = jnp.zeros((L,), jnp.int32)
    bef = jnp.zeros((L,), jnp.int32)
    for t in range(NW):
        row = hvm[t]
        tot = tot + row
        bef = bef + jnp.where(jnp.full((L,), t, jnp.int32) < widv, row, 0)

    padded = ((tot + (T - 1)) >> 8) << 8
    incl = plsc.cumsum(padded)
    mybase = (incl - padded) + bef

    @pl.when(wid == 0)
    def _():
        end_s = jnp.sum(jnp.where(lane == jnp.full((L,), E - 1, jnp.int32),
                                  incl, 0))
        end = jnp.broadcast_to(end_s, (L,))
        for k in range(NBE // L):
            bstart = (jnp.arange(L, dtype=jnp.int32) + L * k) * T
            bcount = jnp.zeros((L,), jnp.int32)
            for e in range(E):
                bnd_s = jnp.sum(jnp.where(lane == jnp.full((L,), e, jnp.int32),
                                          incl, 0))
                bnd = jnp.broadcast_to(bnd_s, (L,))
                bcount = bcount + jnp.where(bstart >= bnd, 1, 0)
            bevm[pl.ds(k * L, L)] = jnp.minimum(bcount, E - 1)
            actvm[pl.ds(k * L, L)] = jnp.where(bstart < end, 1, 0)
        pltpu.sync_copy(bevm, be_hbm)
        pltpu.sync_copy(actvm, act_hbm)

    base = mybase
    idxbufs = [i0, i1, i2, i3, i4, i5, i6, i7]
    for i in range(TOK // L):
        v = evm[pl.ds(i * L, L)]
        gvm[...] = base
        bsel = plsc.load_gather(gvm, [v])
        rank = jnp.zeros((L,), jnp.int32)
        for e in range(E):
            ev = jnp.full((L,), e, jnp.int32)
            m = v == ev
            pref = plsc.cumsum(jnp.where(m, 1, 0))
            rank = jnp.where(m, pref - 1, rank)
            c = jnp.broadcast_to(pref[L - 1], (L,))
            base = base + jnp.where(lane == ev, c, 0)
        posv = bsel + rank
        posflat[pl.ds(i * L, L)] = posv
        idxbufs[i // 2][pl.ds((i % 2) * L, L)] = posv

    aux = [pltpu.async_copy(posflat, pos_hbm.at[pl.ds(tokbase, TOK)], semaux)]
    for j in range(8):
        aux.append(pltpu.async_copy(pvm.at[pl.ds(32 * j, 32)],
                                    ps_hbm.at[idxbufs[j]], semaux))

    bufs = [xbufa, xbufb]
    sin = [semin_a, semin_b]
    sout = [semout_a, semout_b]
    pend_out = [None, None]
    d_in = pltpu.async_copy(x_hbm.at[pl.ds(tokbase, 32)], bufs[0], sin[0])
    for j in range(8):
        b = j % 2
        d_in.wait()
        d_out = pltpu.async_copy(bufs[b], xs_hbm.at[idxbufs[j]], sout[b])
        if j + 1 < 8:
            if pend_out[1 - b] is not None:
                pend_out[1 - b].wait()
            d_in = pltpu.async_copy(
                x_hbm.at[pl.ds(tokbase + 32 * (j + 1), 32)],
                bufs[1 - b], sin[1 - b])
        pend_out[b] = d_out
    pend_out[0].wait()
    pend_out[1].wait()
    for d in aux:
        d.wait()


_DISP_OUT = [
    jax.ShapeDtypeStruct((NPAD, D), jnp.float32),
    jax.ShapeDtypeStruct((NPAD,), jnp.float32),
    jax.ShapeDtypeStruct((N,), jnp.int32),
    jax.ShapeDtypeStruct((NBE,), jnp.int32),
    jax.ShapeDtypeStruct((NBE,), jnp.int32),
]
_DISP_SCRATCH = (
    [
        pltpu.VMEM((TOK,), jnp.int32),
        pltpu.VMEM((NW, L), jnp.int32),
        pltpu.VMEM((L,), jnp.int32),
        pltpu.VMEM((TOK,), jnp.int32),
    ]
    + [pltpu.VMEM((32,), jnp.int32)] * 8
    + [
        pltpu.VMEM((32, D), jnp.float32),
        pltpu.VMEM((32, D), jnp.float32),
        pltpu.VMEM((TOK,), jnp.float32),
        pltpu.VMEM((NBE,), jnp.int32),
        pltpu.VMEM((NBE,), jnp.int32),
        pltpu.SemaphoreType.DMA,
        pltpu.SemaphoreType.DMA,
        pltpu.SemaphoreType.DMA,
        pltpu.SemaphoreType.DMA,
        pltpu.SemaphoreType.DMA,
    ]
)
_sc_dispatch = pl.kernel(_sc_dispatch_body, out_type=_DISP_OUT, mesh=_mesh,
                         compiler_params=_sc_params,
                         scratch_types=_DISP_SCRATCH)


FB = 2048
NF = F // FB


def _glu_partial(x, wg_ref, wu_ref, wd_ref):
    acc = jnp.zeros((T, D), jnp.float32)
    for c in range(FB // FCH):
        wg = wg_ref[0, pl.ds(c * FCH, FCH), :]
        wu = wu_ref[0, pl.ds(c * FCH, FCH), :]
        g = lax.dot_general(x, wg, (((1,), (1,)), ((), ())),
                            preferred_element_type=jnp.float32)
        u = lax.dot_general(x, wu, (((1,), (1,)), ((), ())),
                            preferred_element_type=jnp.float32)
        h = (g * jax.nn.sigmoid(g)) * u
        wd = wd_ref[0, :, pl.ds(c * FCH, FCH)]
        acc = acc + lax.dot_general(h, wd,
                                    (((1,), (1,)), ((), ())),
                                    preferred_element_type=jnp.float32)
    return acc


def _moe_body0(be_ref, act_ref, x_ref, wg_ref, wu_ref, wd_ref, p_ref, o_ref):
    @pl.when(act_ref[pl.program_id(0)] == 1)
    def _():
        o_ref[...] = _glu_partial(x_ref[...], wg_ref, wu_ref, wd_ref) \
            * p_ref[...]


def _moe_body1(be_ref, act_ref, x_ref, wg_ref, wu_ref, wd_ref, p_ref,
               part_ref, o_ref):
    @pl.when(act_ref[pl.program_id(0)] == 1)
    def _():
        o_ref[...] = part_ref[...] + \
            _glu_partial(x_ref[...], wg_ref, wu_ref, wd_ref) * p_ref[...]


_moe_params = pltpu.CompilerParams(
    dimension_semantics=("arbitrary",),
    vmem_limit_bytes=64 * 1024 * 1024,
)


def _moe_mlp(block_expert, act, xs, wg, wu, wd, ps2d):
    def wspecs(fi):
        return [
            pl.BlockSpec((1, FB, D), lambda b, s, a: (s[b], fi, 0)),
            pl.BlockSpec((1, FB, D), lambda b, s, a: (s[b], fi, 0)),
            pl.BlockSpec((1, D, FB), lambda b, s, a: (s[b], 0, fi)),
        ]

    xspec = pl.BlockSpec((T, D), lambda b, s, a: (b, 0))
    pspec = pl.BlockSpec((T, 1), lambda b, s, a: (b, 0))
    part = pl.pallas_call(
        _moe_body0,
        grid_spec=pltpu.PrefetchScalarGridSpec(
            num_scalar_prefetch=2,
            grid=(NB,),
            in_specs=[xspec] + wspecs(0) + [pspec],
            out_specs=xspec,
        ),
        out_shape=jax.ShapeDtypeStruct((NPAD, D), jnp.float32),
        compiler_params=_moe_params,
    )(block_expert, act, xs, wg, wu, wd, ps2d)
    return pl.pallas_call(
        _moe_body1,
        grid_spec=pltpu.PrefetchScalarGridSpec(
            num_scalar_prefetch=2,
            grid=(NB,),
            in_specs=[xspec] + wspecs(1) + [pspec, xspec],
            out_specs=xspec,
        ),
        out_shape=jax.ShapeDtypeStruct((NPAD, D), jnp.float32),
        compiler_params=_moe_params,
    )(block_expert, act, xs, wg, wu, wd, ps2d, part)


def _sc_unpermute_body(os_hbm, pos_hbm, y_hbm,
                       i0, i1, i2, i3, i4, i5, i6, i7,
                       posvm, bufa, bufb,
                       semin_a, semin_b, semout_a, semout_b):
    tokbase = _wid() * TOK
    pltpu.sync_copy(pos_hbm.at[pl.ds(tokbase, TOK)], posvm)
    idxbufs = [i0, i1, i2, i3, i4, i5, i6, i7]
    for j in range(8):
        for q in range(2):
            idxbufs[j][pl.ds(q * L, L)] = posvm[pl.ds(32 * j + q * L, L)]
    bufs = [bufa, bufb]
    sin = [semin_a, semin_b]
    sout = [semout_a, semout_b]
    pend_out = [None, None]
    d_in = pltpu.async_copy(os_hbm.at[idxbufs[0]], bufs[0], sin[0])
    for j in range(8):
        b = j % 2
        d_in.wait()
        d_out = pltpu.async_copy(
            bufs[b], y_hbm.at[pl.ds(tokbase + 32 * j, 32)], sout[b])
        if j + 1 < 8:
            if pend_out[1 - b] is not None:
                pend_out[1 - b].wait()
            d_in = pltpu.async_copy(os_hbm.at[idxbufs[j + 1]],
                                    bufs[1 - b], sin[1 - b])
        pend_out[b] = d_out
    pend_out[0].wait()
    pend_out[1].wait()


_UNP_OUT = jax.ShapeDtypeStruct((N, D), jnp.float32)
_UNP_SCRATCH = (
    [pltpu.VMEM((32,), jnp.int32)] * 8
    + [
        pltpu.VMEM((TOK,), jnp.int32),
        pltpu.VMEM((32, D), jnp.float32),
        pltpu.VMEM((32, D), jnp.float32),
        pltpu.SemaphoreType.DMA,
        pltpu.SemaphoreType.DMA,
        pltpu.SemaphoreType.DMA,
        pltpu.SemaphoreType.DMA,
    ]
)
_sc_unpermute = pl.kernel(_sc_unpermute_body, out_type=_UNP_OUT, mesh=_mesh,
                          compiler_params=_sc_params,
                          scratch_types=_UNP_SCRATCH)


def kernel(hidden_states, W_router, W_gate, W_up, W_down):
    x2d = hidden_states.reshape(N, D)
    prob2d, exp2d, hist3 = _router(x2d, W_router)
    expert = exp2d.reshape(N)
    prob = prob2d.reshape(N)

    xs, ps, pos, block_expert, act = _sc_dispatch(
        expert, prob, x2d, hist3.reshape(NW, L))

    out_sorted = _moe_mlp(block_expert, act, xs, W_gate, W_up, W_down,
                          ps.reshape(NPAD, 1))

    y = _sc_unpermute(out_sorted, pos)
    return y.reshape(B, S, D)
